# baseline (speedup 1.0000x reference)
import functools

import numpy as np

import jax
import jax.numpy as jnp
from jax import lax
from jax.experimental import pallas as pl
from jax.experimental.pallas import tpu as pltpu
from jax.experimental.pallas import tpu_sc as plsc

NUM_V = 100000
NUM_F = 200000

NC = 2
NS = 16
NW = NC * NS
L = 16

FT = 6272
FP = NW * FT
DUMMY = NUM_V

VT = 6256
VP = NS * VT


def _sc_body(vt3, ft3, t_out, c_out, s_out, s_fin,
             t_sh, c_sh, cx, cy, cz, sbuf, zbuf, obuf,
             ii, ij, ik, gi, gj, gk, tbuf,
             sem_i, sem_v, sem_z, sem_g, sem_s):
    cid = lax.axis_index("c")
    sid = lax.axis_index("s")
    wid = cid * NS + sid
    vbase = sid * VT

    di0 = pltpu.async_copy(ft3.at[wid * 3 + 0], ii, sem_i)
    di1 = pltpu.async_copy(ft3.at[wid * 3 + 1], ij, sem_i)
    di2 = pltpu.async_copy(ft3.at[wid * 3 + 2], ik, sem_i)
    dv0 = pltpu.async_copy(vt3.at[sid], cx, sem_v)
    dv1 = pltpu.async_copy(vt3.at[NS + sid], cy, sem_v)
    dv2 = pltpu.async_copy(vt3.at[2 * NS + sid], cz, sem_v)

    def o_step(n, _):
        obuf[pl.ds(n * L, L)] = jnp.ones((L,), jnp.float32)
        return 0

    lax.fori_loop(0, FT // L, o_step, 0)

    dv0.wait()
    dv1.wait()
    dv2.wait()

    def s_step(n, _):
        sl = pl.ds(n * L, L)
        sbuf[sl] = cx[sl] + cy[sl] + cz[sl]
        zbuf[sl] = jnp.zeros((L,), jnp.float32)
        return 0

    lax.fori_loop(0, VT // L, s_step, 0)

    ds_s = pltpu.async_copy(sbuf, s_out.at[pl.ds(cid * VP + vbase, VT)], sem_v)
    dz1 = pltpu.async_copy(zbuf, t_sh.at[pl.ds(vbase, VT)], sem_z)
    dz2 = pltpu.async_copy(zbuf, c_sh.at[pl.ds(vbase, VT)], sem_z)
    ds_s.wait()
    di0.wait()
    di1.wait()
    di2.wait()
    dz1.wait()
    dz2.wait()
    plsc.subcore_barrier()

    s_row = s_out.at[pl.ds(cid * VP, VP)]
    dg0 = pltpu.async_copy(s_row.at[ii], gi, sem_g)
    dg1 = pltpu.async_copy(s_row.at[ij], gj, sem_g)
    dg2 = pltpu.async_copy(s_row.at[ik], gk, sem_g)
    ds3 = pltpu.async_copy(obuf, c_sh.at[ii], sem_s, add=True)
    ds4 = pltpu.async_copy(obuf, c_sh.at[ij], sem_s, add=True)
    ds5 = pltpu.async_copy(obuf, c_sh.at[ik], sem_s, add=True)
    dg0.wait()
    dg1.wait()
    dg2.wait()

    def t_step(n, _):
        sl = pl.ds(n * L, L)
        tbuf[sl] = gi[sl] + gj[sl] + gk[sl]
        return 0

    lax.fori_loop(0, FT // L, t_step, 0)

    ds0 = pltpu.async_copy(tbuf, t_sh.at[ii], sem_s, add=True)
    ds1 = pltpu.async_copy(tbuf, t_sh.at[ij], sem_s, add=True)
    ds2 = pltpu.async_copy(tbuf, t_sh.at[ik], sem_s, add=True)
    ds0.wait()
    ds1.wait()
    ds2.wait()
    ds3.wait()
    ds4.wait()
    ds5.wait()
    plsc.subcore_barrier()

    dc0 = pltpu.async_copy(t_sh.at[pl.ds(vbase, VT)], cx, sem_g)
    dc1 = pltpu.async_copy(c_sh.at[pl.ds(vbase, VT)], cy, sem_g)
    dc0.wait()
    do0 = pltpu.async_copy(cx, t_out.at[cid, sid], sem_s)
    dc1.wait()
    do1 = pltpu.async_copy(cy, c_out.at[cid, sid], sem_s)
    do0.wait()
    do1.wait()

    @pl.when(cid == 0)
    def _():
        pltpu.sync_copy(sbuf, s_fin.at[sid])


_sc_call = pl.kernel(
    _sc_body,
    out_type=[
        jax.ShapeDtypeStruct((NC, NS, VT), jnp.float32),
        jax.ShapeDtypeStruct((NC, NS, VT), jnp.float32),
        jax.ShapeDtypeStruct((NC * VP,), jnp.float32),
        jax.ShapeDtypeStruct((NS, VT), jnp.float32),
    ],
    mesh=plsc.VectorSubcoreMesh(core_axis_name="c", subcore_axis_name="s",
                                num_cores=NC, num_subcores=NS),
    scratch_types=[
        pltpu.VMEM_SHARED((VP,), jnp.float32),
        pltpu.VMEM_SHARED((VP,), jnp.float32),
        pltpu.VMEM((VT,), jnp.float32),
        pltpu.VMEM((VT,), jnp.float32),
        pltpu.VMEM((VT,), jnp.float32),
        pltpu.VMEM((VT,), jnp.float32),
        pltpu.VMEM((VT,), jnp.float32),
        pltpu.VMEM((FT,), jnp.float32),
        pltpu.VMEM((FT,), jnp.int32),
        pltpu.VMEM((FT,), jnp.int32),
        pltpu.VMEM((FT,), jnp.int32),
        pltpu.VMEM((FT,), jnp.float32),
        pltpu.VMEM((FT,), jnp.float32),
        pltpu.VMEM((FT,), jnp.float32),
        pltpu.VMEM((FT,), jnp.float32),
        pltpu.SemaphoreType.DMA,
        pltpu.SemaphoreType.DMA,
        pltpu.SemaphoreType.DMA,
        pltpu.SemaphoreType.DMA,
        pltpu.SemaphoreType.DMA,
    ],
)


def _finalize_body(tp_ref, cp_ref, s_ref, out_ref):
    t = tp_ref[0] + tp_ref[1]
    c = cp_ref[0] + cp_ref[1]
    s = s_ref[...]
    deg = jnp.maximum(2.0 * c, 1.0)
    term = (t - c * s) / deg
    val = (jnp.sum(s) - jnp.sum(term)) / jnp.float32(3 * NUM_V)
    out_ref[...] = jnp.broadcast_to(val, (1, 1))


def kernel(verts, faces):
    vt3 = jnp.pad(verts, ((0, VP - NUM_V), (0, 0))).T.reshape(3 * NS, VT)
    dummy_faces = jnp.asarray(
        DUMMY + (np.arange((FP - NUM_F) * 3, dtype=np.int32)
                 % (VP - NUM_V)).reshape(FP - NUM_F, 3))
    ft3 = jnp.concatenate([faces, dummy_faces]).reshape(NW, FT, 3
                  ).transpose(0, 2, 1).reshape(NW * 3, FT)

    t_p, c_p, _s_gather, s_p = _sc_call(vt3, ft3)

    out = pl.pallas_call(
        _finalize_body,
        out_shape=jax.ShapeDtypeStruct((1, 1), jnp.float32),
    )(t_p, c_p, s_p)
    return out[0, 0]


# speedup vs baseline: 1.5222x; 1.5222x over previous
import functools

import numpy as np

import jax
import jax.numpy as jnp
from jax import lax
from jax.experimental import pallas as pl
from jax.experimental.pallas import tpu as pltpu
from jax.experimental.pallas import tpu_sc as plsc

NUM_V = 100000
NUM_F = 200000

NC = 2
NS = 16
NW = NC * NS
L = 16

FT = 6272
FP = NW * FT
DUMMY = NUM_V

VT = 6256
VP = NS * VT


def _sc_body(vt3, ft3, t_out, c_out, s_out,
             s_sh, t_sh, c_sh, cx, cy, cz, sbuf, zbuf, obuf,
             ii, ij, ik, gi, gj, gk, tbuf,
             sem_i, sem_v, sem_z, sem_g, sem_s):
    cid = lax.axis_index("c")
    sid = lax.axis_index("s")
    wid = cid * NS + sid
    vbase = sid * VT

    fb = wid * FT
    di0 = pltpu.async_copy(ft3.at[pl.ds(fb, FT)], ii, sem_i)
    di1 = pltpu.async_copy(ft3.at[pl.ds(FP + fb, FT)], ij, sem_i)
    di2 = pltpu.async_copy(ft3.at[pl.ds(2 * FP + fb, FT)], ik, sem_i)
    dv0 = pltpu.async_copy(vt3.at[sid], cx, sem_v)
    dv1 = pltpu.async_copy(vt3.at[NS + sid], cy, sem_v)
    dv2 = pltpu.async_copy(vt3.at[2 * NS + sid], cz, sem_v)

    def o_step(n, _):
        obuf[pl.ds(n * L, L)] = jnp.ones((L,), jnp.float32)
        return 0

    lax.fori_loop(0, FT // L, o_step, 0)

    dv0.wait()
    dv1.wait()
    dv2.wait()

    def s_step(n, _):
        sl = pl.ds(n * L, L)
        sbuf[sl] = cx[sl] + cy[sl] + cz[sl]
        zbuf[sl] = jnp.zeros((L,), jnp.float32)
        return 0

    lax.fori_loop(0, VT // L, s_step, 0)

    ds_s = pltpu.async_copy(sbuf, s_sh.at[pl.ds(vbase, VT)], sem_v)
    dz1 = pltpu.async_copy(zbuf, t_sh.at[pl.ds(vbase, VT)], sem_z)
    dz2 = pltpu.async_copy(zbuf, c_sh.at[pl.ds(vbase, VT)], sem_z)
    ds_s.wait()
    di0.wait()
    di1.wait()
    di2.wait()
    dz1.wait()
    dz2.wait()
    plsc.subcore_barrier()

    dg0 = pltpu.async_copy(s_sh.at[ii], gi, sem_g)
    dg1 = pltpu.async_copy(s_sh.at[ij], gj, sem_g)
    dg2 = pltpu.async_copy(s_sh.at[ik], gk, sem_g)
    ds3 = pltpu.async_copy(obuf, c_sh.at[ii], sem_s, add=True)
    ds4 = pltpu.async_copy(obuf, c_sh.at[ij], sem_s, add=True)
    ds5 = pltpu.async_copy(obuf, c_sh.at[ik], sem_s, add=True)
    dg0.wait()
    dg1.wait()
    dg2.wait()

    def t_step(n, _):
        sl = pl.ds(n * L, L)
        tbuf[sl] = gi[sl] + gj[sl] + gk[sl]
        return 0

    lax.fori_loop(0, FT // L, t_step, 0)

    ds0 = pltpu.async_copy(tbuf, t_sh.at[ii], sem_s, add=True)
    ds1 = pltpu.async_copy(tbuf, t_sh.at[ij], sem_s, add=True)
    ds2 = pltpu.async_copy(tbuf, t_sh.at[ik], sem_s, add=True)
    ds0.wait()
    ds1.wait()
    ds2.wait()
    ds3.wait()
    ds4.wait()
    ds5.wait()
    plsc.subcore_barrier()

    dc0 = pltpu.async_copy(t_sh.at[pl.ds(vbase, VT)], cx, sem_g)
    dc1 = pltpu.async_copy(c_sh.at[pl.ds(vbase, VT)], cy, sem_g)
    dc0.wait()
    do0 = pltpu.async_copy(cx, t_out.at[cid, sid], sem_s)
    dc1.wait()
    do1 = pltpu.async_copy(cy, c_out.at[cid, sid], sem_s)
    do0.wait()
    do1.wait()

    @pl.when(cid == 0)
    def _():
        pltpu.sync_copy(sbuf, s_out.at[sid])


_sc_call = pl.kernel(
    _sc_body,
    out_type=[
        jax.ShapeDtypeStruct((NC, NS, VT), jnp.float32),
        jax.ShapeDtypeStruct((NC, NS, VT), jnp.float32),
        jax.ShapeDtypeStruct((NS, VT), jnp.float32),
    ],
    mesh=plsc.VectorSubcoreMesh(core_axis_name="c", subcore_axis_name="s",
                                num_cores=NC, num_subcores=NS),
    scratch_types=[
        pltpu.VMEM_SHARED((VP,), jnp.float32),
        pltpu.VMEM_SHARED((VP,), jnp.float32),
        pltpu.VMEM_SHARED((VP,), jnp.float32),
        pltpu.VMEM((VT,), jnp.float32),
        pltpu.VMEM((VT,), jnp.float32),
        pltpu.VMEM((VT,), jnp.float32),
        pltpu.VMEM((VT,), jnp.float32),
        pltpu.VMEM((VT,), jnp.float32),
        pltpu.VMEM((FT,), jnp.float32),
        pltpu.VMEM((FT,), jnp.int32),
        pltpu.VMEM((FT,), jnp.int32),
        pltpu.VMEM((FT,), jnp.int32),
        pltpu.VMEM((FT,), jnp.float32),
        pltpu.VMEM((FT,), jnp.float32),
        pltpu.VMEM((FT,), jnp.float32),
        pltpu.VMEM((FT,), jnp.float32),
        pltpu.SemaphoreType.DMA,
        pltpu.SemaphoreType.DMA,
        pltpu.SemaphoreType.DMA,
        pltpu.SemaphoreType.DMA,
        pltpu.SemaphoreType.DMA,
    ],
)


def _finalize_body(tp_ref, cp_ref, s_ref, out_ref):
    t = tp_ref[0] + tp_ref[1]
    c = cp_ref[0] + cp_ref[1]
    s = s_ref[...]
    deg = jnp.maximum(2.0 * c, 1.0)
    term = (t - c * s) / deg
    val = (jnp.sum(s) - jnp.sum(term)) / jnp.float32(3 * NUM_V)
    out_ref[...] = jnp.broadcast_to(val, (1, 1))


def kernel(verts, faces):
    vt3 = jnp.pad(verts, ((0, VP - NUM_V), (0, 0))).T.reshape(3 * NS, VT)
    dummy_faces = jnp.asarray(
        DUMMY + (np.arange((FP - NUM_F) * 3, dtype=np.int32)
                 % (VP - NUM_V)).reshape(3, FP - NUM_F))
    ft3 = jnp.concatenate([faces.T, dummy_faces], axis=1).reshape(3 * FP)

    t_p, c_p, s_p = _sc_call(vt3, ft3)

    out = pl.pallas_call(
        _finalize_body,
        out_shape=jax.ShapeDtypeStruct((1, 1), jnp.float32),
    )(t_p, c_p, s_p)
    return out[0, 0]


# speedup vs baseline: 1.5540x; 1.0209x over previous
import functools

import numpy as np

import jax
import jax.numpy as jnp
from jax import lax
from jax.experimental import pallas as pl
from jax.experimental.pallas import tpu as pltpu
from jax.experimental.pallas import tpu_sc as plsc

NUM_V = 100000
NUM_F = 200000

NC = 2
NS = 16
NW = NC * NS
L = 16

FT = 6272
FP = NW * FT
DUMMY = NUM_V

VT = 6256
VP = NS * VT


def _sc_body(vt3, ft3, t_out, c_out, s_out,
             s_sh, t_sh, c_sh, cx, cy, cz, sbuf, zbuf, obuf,
             ii, ij, ik, gi, gj, gk, tbuf,
             sem_i, sem_v, sem_z, sem_g, sem_s):
    cid = lax.axis_index("c")
    sid = lax.axis_index("s")
    wid = cid * NS + sid
    vbase = sid * VT

    di0 = pltpu.async_copy(ft3.at[wid * 3 + 0], ii, sem_i)
    di1 = pltpu.async_copy(ft3.at[wid * 3 + 1], ij, sem_i)
    di2 = pltpu.async_copy(ft3.at[wid * 3 + 2], ik, sem_i)
    dv0 = pltpu.async_copy(vt3.at[sid], cx, sem_v)
    dv1 = pltpu.async_copy(vt3.at[NS + sid], cy, sem_v)
    dv2 = pltpu.async_copy(vt3.at[2 * NS + sid], cz, sem_v)

    def o_step(n, _):
        obuf[pl.ds(n * L, L)] = jnp.ones((L,), jnp.float32)
        return 0

    lax.fori_loop(0, FT // L, o_step, 0)

    dv0.wait()
    dv1.wait()
    dv2.wait()

    def s_step(n, _):
        sl = pl.ds(n * L, L)
        sbuf[sl] = cx[sl] + cy[sl] + cz[sl]
        zbuf[sl] = jnp.zeros((L,), jnp.float32)
        return 0

    lax.fori_loop(0, VT // L, s_step, 0)

    ds_s = pltpu.async_copy(sbuf, s_sh.at[pl.ds(vbase, VT)], sem_v)
    dz1 = pltpu.async_copy(zbuf, t_sh.at[pl.ds(vbase, VT)], sem_z)
    dz2 = pltpu.async_copy(zbuf, c_sh.at[pl.ds(vbase, VT)], sem_z)
    ds_s.wait()
    di0.wait()
    di1.wait()
    di2.wait()
    dz1.wait()
    dz2.wait()
    plsc.subcore_barrier()

    dg0 = pltpu.async_copy(s_sh.at[ii], gi, sem_g)
    dg1 = pltpu.async_copy(s_sh.at[ij], gj, sem_g)
    dg2 = pltpu.async_copy(s_sh.at[ik], gk, sem_g)
    ds3 = pltpu.async_copy(obuf, c_sh.at[ii], sem_s, add=True)
    ds4 = pltpu.async_copy(obuf, c_sh.at[ij], sem_s, add=True)
    ds5 = pltpu.async_copy(obuf, c_sh.at[ik], sem_s, add=True)
    dg0.wait()
    dg1.wait()
    dg2.wait()

    def t_step(n, _):
        for u in range(4):
            sl = pl.ds((n * 4 + u) * L, L)
            tbuf[sl] = gi[sl] + gj[sl] + gk[sl]
        return 0

    lax.fori_loop(0, FT // (4 * L), t_step, 0)

    ds0 = pltpu.async_copy(tbuf, t_sh.at[ii], sem_s, add=True)
    ds1 = pltpu.async_copy(tbuf, t_sh.at[ij], sem_s, add=True)
    ds2 = pltpu.async_copy(tbuf, t_sh.at[ik], sem_s, add=True)
    ds0.wait()
    ds1.wait()
    ds2.wait()
    ds3.wait()
    ds4.wait()
    ds5.wait()
    plsc.subcore_barrier()

    dc0 = pltpu.async_copy(t_sh.at[pl.ds(vbase, VT)], cx, sem_g)
    dc1 = pltpu.async_copy(c_sh.at[pl.ds(vbase, VT)], cy, sem_g)
    dc0.wait()
    do0 = pltpu.async_copy(cx, t_out.at[cid, sid], sem_s)
    dc1.wait()
    do1 = pltpu.async_copy(cy, c_out.at[cid, sid], sem_s)
    do0.wait()
    do1.wait()

    @pl.when(cid == 0)
    def _():
        pltpu.sync_copy(sbuf, s_out.at[sid])


_sc_call = pl.kernel(
    _sc_body,
    out_type=[
        jax.ShapeDtypeStruct((NC, NS, VT), jnp.float32),
        jax.ShapeDtypeStruct((NC, NS, VT), jnp.float32),
        jax.ShapeDtypeStruct((NS, VT), jnp.float32),
    ],
    mesh=plsc.VectorSubcoreMesh(core_axis_name="c", subcore_axis_name="s",
                                num_cores=NC, num_subcores=NS),
    scratch_types=[
        pltpu.VMEM_SHARED((VP,), jnp.float32),
        pltpu.VMEM_SHARED((VP,), jnp.float32),
        pltpu.VMEM_SHARED((VP,), jnp.float32),
        pltpu.VMEM((VT,), jnp.float32),
        pltpu.VMEM((VT,), jnp.float32),
        pltpu.VMEM((VT,), jnp.float32),
        pltpu.VMEM((VT,), jnp.float32),
        pltpu.VMEM((VT,), jnp.float32),
        pltpu.VMEM((FT,), jnp.float32),
        pltpu.VMEM((FT,), jnp.int32),
        pltpu.VMEM((FT,), jnp.int32),
        pltpu.VMEM((FT,), jnp.int32),
        pltpu.VMEM((FT,), jnp.float32),
        pltpu.VMEM((FT,), jnp.float32),
        pltpu.VMEM((FT,), jnp.float32),
        pltpu.VMEM((FT,), jnp.float32),
        pltpu.SemaphoreType.DMA,
        pltpu.SemaphoreType.DMA,
        pltpu.SemaphoreType.DMA,
        pltpu.SemaphoreType.DMA,
        pltpu.SemaphoreType.DMA,
    ],
)


def _finalize_body(tp_ref, cp_ref, s_ref, out_ref):
    t = tp_ref[0] + tp_ref[1]
    c = cp_ref[0] + cp_ref[1]
    s = s_ref[...]
    deg = jnp.maximum(2.0 * c, 1.0)
    term = (t - c * s) / deg
    val = (jnp.sum(s) - jnp.sum(term)) / jnp.float32(3 * NUM_V)
    out_ref[...] = jnp.broadcast_to(val, (1, 1))


def kernel(verts, faces):
    vt3 = jnp.pad(verts, ((0, VP - NUM_V), (0, 0))).T.reshape(3 * NS, VT)
    dummy_faces = jnp.asarray(
        DUMMY + (np.arange((FP - NUM_F) * 3, dtype=np.int32)
                 % (VP - NUM_V)).reshape(FP - NUM_F, 3))
    ft3 = jnp.concatenate([faces, dummy_faces]).reshape(NW, FT, 3
                  ).transpose(0, 2, 1).reshape(NW * 3, FT)

    t_p, c_p, s_p = _sc_call(vt3, ft3)

    out = pl.pallas_call(
        _finalize_body,
        out_shape=jax.ShapeDtypeStruct((1, 1), jnp.float32),
    )(t_p, c_p, s_p)
    return out[0, 0]


# speedup vs baseline: 1.5701x; 1.0103x over previous
import functools

import numpy as np

import jax
import jax.numpy as jnp
from jax import lax
from jax.experimental import pallas as pl
from jax.experimental.pallas import tpu as pltpu
from jax.experimental.pallas import tpu_sc as plsc

NUM_V = 100000
NUM_F = 200000

NC = 2
NS = 16
NW = NC * NS
L = 16

FT = 6272
FP = NW * FT
DUMMY = NUM_V

VT = 6256
VP = NS * VT


def _sc_body(vt3, ft3, t_out, c_out, s_out,
             s_sh, t_sh, c_sh, cx, cy, cz, sbuf, zbuf, obuf,
             ii, ij, ik, gi, gj, gk, tbuf,
             sem_i, sem_v, sem_z, sem_g, sem_s):
    cid = lax.axis_index("c")
    sid = lax.axis_index("s")
    wid = cid * NS + sid
    vbase = sid * VT

    di0 = pltpu.async_copy(ft3.at[wid * 3 + 0], ii, sem_i)
    di1 = pltpu.async_copy(ft3.at[wid * 3 + 1], ij, sem_i)
    di2 = pltpu.async_copy(ft3.at[wid * 3 + 2], ik, sem_i)
    dv0 = pltpu.async_copy(vt3.at[sid], cx, sem_v)
    dv1 = pltpu.async_copy(vt3.at[NS + sid], cy, sem_v)
    dv2 = pltpu.async_copy(vt3.at[2 * NS + sid], cz, sem_v)

    def o_step(n, _):
        for u in range(4):
            obuf[pl.ds((n * 4 + u) * L, L)] = jnp.ones((L,), jnp.float32)
        return 0

    lax.fori_loop(0, FT // (4 * L), o_step, 0)

    dv0.wait()
    dv1.wait()
    dv2.wait()

    def s_step(n, _):
        for u in range(4):
            sl = pl.ds((n * 4 + u) * L, L)
            sbuf[sl] = cx[sl] + cy[sl] + cz[sl]
            zbuf[sl] = jnp.zeros((L,), jnp.float32)
        return 0

    lax.fori_loop(0, VT // (4 * L), s_step, 0)
    for u in range(VT // (4 * L) * 4, VT // L):
        sl = pl.ds(u * L, L)
        sbuf[sl] = cx[sl] + cy[sl] + cz[sl]
        zbuf[sl] = jnp.zeros((L,), jnp.float32)

    ds_s = pltpu.async_copy(sbuf, s_sh.at[pl.ds(vbase, VT)], sem_v)
    dz1 = pltpu.async_copy(zbuf, t_sh.at[pl.ds(vbase, VT)], sem_z)
    dz2 = pltpu.async_copy(zbuf, c_sh.at[pl.ds(vbase, VT)], sem_z)
    ds_s.wait()
    di0.wait()
    di1.wait()
    di2.wait()
    dz1.wait()
    dz2.wait()
    plsc.subcore_barrier()

    dg0 = pltpu.async_copy(s_sh.at[ii], gi, sem_g)
    dg1 = pltpu.async_copy(s_sh.at[ij], gj, sem_g)
    dg2 = pltpu.async_copy(s_sh.at[ik], gk, sem_g)
    ds3 = pltpu.async_copy(obuf, c_sh.at[ii], sem_s, add=True)
    ds4 = pltpu.async_copy(obuf, c_sh.at[ij], sem_s, add=True)
    ds5 = pltpu.async_copy(obuf, c_sh.at[ik], sem_s, add=True)
    dg0.wait()
    dg1.wait()
    dg2.wait()

    def t_step(n, _):
        for u in range(4):
            sl = pl.ds((n * 4 + u) * L, L)
            tbuf[sl] = gi[sl] + gj[sl] + gk[sl]
        return 0

    lax.fori_loop(0, FT // (4 * L), t_step, 0)

    ds0 = pltpu.async_copy(tbuf, t_sh.at[ii], sem_s, add=True)
    ds1 = pltpu.async_copy(tbuf, t_sh.at[ij], sem_s, add=True)
    ds2 = pltpu.async_copy(tbuf, t_sh.at[ik], sem_s, add=True)
    ds0.wait()
    ds1.wait()
    ds2.wait()
    ds3.wait()
    ds4.wait()
    ds5.wait()
    plsc.subcore_barrier()

    dc0 = pltpu.async_copy(t_sh.at[pl.ds(vbase, VT)], cx, sem_g)
    dc1 = pltpu.async_copy(c_sh.at[pl.ds(vbase, VT)], cy, sem_g)
    dc0.wait()
    do0 = pltpu.async_copy(cx, t_out.at[cid, sid], sem_s)
    dc1.wait()
    do1 = pltpu.async_copy(cy, c_out.at[cid, sid], sem_s)
    do0.wait()
    do1.wait()

    @pl.when(cid == 0)
    def _():
        pltpu.sync_copy(sbuf, s_out.at[sid])


_sc_call = pl.kernel(
    _sc_body,
    out_type=[
        jax.ShapeDtypeStruct((NC, NS, VT), jnp.float32),
        jax.ShapeDtypeStruct((NC, NS, VT), jnp.float32),
        jax.ShapeDtypeStruct((NS, VT), jnp.float32),
    ],
    mesh=plsc.VectorSubcoreMesh(core_axis_name="c", subcore_axis_name="s",
                                num_cores=NC, num_subcores=NS),
    scratch_types=[
        pltpu.VMEM_SHARED((VP,), jnp.float32),
        pltpu.VMEM_SHARED((VP,), jnp.float32),
        pltpu.VMEM_SHARED((VP,), jnp.float32),
        pltpu.VMEM((VT,), jnp.float32),
        pltpu.VMEM((VT,), jnp.float32),
        pltpu.VMEM((VT,), jnp.float32),
        pltpu.VMEM((VT,), jnp.float32),
        pltpu.VMEM((VT,), jnp.float32),
        pltpu.VMEM((FT,), jnp.float32),
        pltpu.VMEM((FT,), jnp.int32),
        pltpu.VMEM((FT,), jnp.int32),
        pltpu.VMEM((FT,), jnp.int32),
        pltpu.VMEM((FT,), jnp.float32),
        pltpu.VMEM((FT,), jnp.float32),
        pltpu.VMEM((FT,), jnp.float32),
        pltpu.VMEM((FT,), jnp.float32),
        pltpu.SemaphoreType.DMA,
        pltpu.SemaphoreType.DMA,
        pltpu.SemaphoreType.DMA,
        pltpu.SemaphoreType.DMA,
        pltpu.SemaphoreType.DMA,
    ],
)


def _finalize_body(tp_ref, cp_ref, s_ref, out_ref):
    t = tp_ref[0] + tp_ref[1]
    c = cp_ref[0] + cp_ref[1]
    s = s_ref[...]
    deg = jnp.maximum(2.0 * c, 1.0)
    term = (t - c * s) / deg
    val = (jnp.sum(s) - jnp.sum(term)) / jnp.float32(3 * NUM_V)
    out_ref[...] = jnp.broadcast_to(val, (1, 1))


def kernel(verts, faces):
    vt3 = jnp.pad(verts, ((0, VP - NUM_V), (0, 0))).T.reshape(3 * NS, VT)
    dummy_faces = jnp.asarray(
        DUMMY + (np.arange((FP - NUM_F) * 3, dtype=np.int32)
                 % (VP - NUM_V)).reshape(FP - NUM_F, 3))
    ft3 = jnp.concatenate([faces, dummy_faces]).reshape(NW, FT, 3
                  ).transpose(0, 2, 1).reshape(NW * 3, FT)

    t_p, c_p, s_p = _sc_call(vt3, ft3)

    out = pl.pallas_call(
        _finalize_body,
        out_shape=jax.ShapeDtypeStruct((1, 1), jnp.float32),
    )(t_p, c_p, s_p)
    return out[0, 0]
